# baseline (speedup 1.0000x reference)
import functools

import jax
import jax.numpy as jnp
from jax import lax
from jax.experimental import pallas as pl
from jax.experimental.pallas import tpu as pltpu
from jax.experimental.pallas import tpu_sc as plsc

_PITCH_MEAN = 150.0
_PITCH_STD = 50.0

_NC = 2
_NS = 16
_NW = _NC * _NS
_LANES = 16
_CH = 64
_RBUF = 4
_WBUF = 3


def _tc_prelude_body(pt_ref, st_ref, spk_ref, pw_ref, pb_ref, ow_ref, ob_ref,
                     ph_ref, pit_ref, tab_ref, rr_ref, fp_ref, ph2_ref,
                     pn2_ref):
    ow = ow_ref[...]
    prec = lax.Precision.HIGHEST
    tab_ref[...] = jnp.dot(pt_ref[...], ow, precision=prec,
                           preferred_element_type=jnp.float32)
    n_spk = st_ref.shape[0]
    b = spk_ref.shape[0]
    oh = (spk_ref[...] == lax.broadcasted_iota(jnp.int32, (b, n_spk), 1)
          ).astype(jnp.float32)
    srows = jnp.dot(oh, st_ref[...], precision=prec,
                    preferred_element_type=jnp.float32)
    const = jnp.dot(pb_ref[...], ow, precision=prec,
                    preferred_element_type=jnp.float32) + ob_ref[...]
    rr_ref[...] = jnp.dot(srows, ow, precision=prec,
                          preferred_element_type=jnp.float32) + const
    fp_ref[...] = jnp.broadcast_to(
        jnp.dot(pw_ref[...], ow, precision=prec,
                preferred_element_type=jnp.float32), fp_ref.shape)
    ph2_ref[...] = ph_ref[...].reshape(ph2_ref.shape)
    pn2_ref[...] = ((jnp.exp(pit_ref[...]) - _PITCH_MEAN)
                    / _PITCH_STD).reshape(pn2_ref.shape)


def _sc_embed_body(ph2_hbm, pn2_hbm, tab_hbm, rr_hbm, fp_hbm, out_hbm,
                   idx_v, pn_v, rows_v, obuf_v, fp_v, r_v, gsem, osem, csem,
                   tpw):
    cid = lax.axis_index("c")
    sid = lax.axis_index("s")
    wid = sid * _NC + cid
    base = wid * tpw
    n_chunks = tpw // _CH
    nrows = tpw // 128
    row0 = wid * nrows
    cpr = 128 // _CH
    b = base // (out_hbm.shape[0] // rr_hbm.shape[0])

    pltpu.sync_copy(ph2_hbm.at[pl.ds(row0, nrows)], idx_v)

    def gdesc(c):
        s = lax.rem(c, _RBUF)
        return pltpu.make_async_copy(
            tab_hbm.at[idx_v.at[c // cpr, pl.ds((c % cpr) * _CH, _CH)]],
            rows_v.at[s], gsem.at[s])

    def odesc(c):
        o = lax.rem(c, _WBUF)
        return pltpu.make_async_copy(
            obuf_v.at[o], out_hbm.at[pl.ds(base + c * _CH, _CH)], osem.at[o])

    for c0 in range(_RBUF):
        gdesc(c0).start()

    c_r = pltpu.async_copy(rr_hbm.at[b], r_v, csem)
    c_f = pltpu.async_copy(fp_hbm.at[0], fp_v, csem)
    c_p = pltpu.async_copy(pn2_hbm.at[pl.ds(row0, nrows)], pn_v, csem)
    c_r.wait()
    c_f.wait()
    c_p.wait()

    nvec = 256 // _LANES
    fp_regs = [fp_v[pl.ds(j * _LANES, _LANES)] for j in range(nvec)]
    r_regs = [r_v[pl.ds(j * _LANES, _LANES)] for j in range(nvec)]

    def chunk_body(c, carry):
        s = lax.rem(c, _RBUF)
        o = lax.rem(c, _WBUF)

        @pl.when(c >= _WBUF)
        def _drain():
            odesc(c - _WBUF).wait()

        gdesc(c).wait()

        def group(g, carry2):
            blk = pn_v[c // cpr, pl.ds((c % cpr) * _CH + g * _LANES, _LANES)]
            for k in range(_LANES):
                idx = jnp.full((_LANES, 1), k, jnp.int32)
                spl = lax.gather(
                    blk, idx,
                    lax.GatherDimensionNumbers(
                        offset_dims=(), collapsed_slice_dims=(0,),
                        start_index_map=(0,)),
                    slice_sizes=(1,),
                    mode=lax.GatherScatterMode.PROMISE_IN_BOUNDS)
                row = g * _LANES + k
                for j in range(nvec):
                    sl = pl.ds(j * _LANES, _LANES)
                    obuf_v[o, row, sl] = (rows_v[s, row, sl]
                                          + spl * fp_regs[j] + r_regs[j])
            return carry2

        lax.fori_loop(0, _CH // _LANES, group, 0)
        odesc(c).start()

        @pl.when(c + _RBUF < n_chunks)
        def _prefetch():
            gdesc(c + _RBUF).start()

        return carry

    lax.fori_loop(0, n_chunks, chunk_body, 0)
    for t in range(min(_WBUF, n_chunks)):
        odesc(n_chunks - 1 - t).wait()


def kernel(phonemes, pitches, speakers, phoneme_table, speaker_table,
           pitch_w, pitch_b, out_w, out_b):
    B, T = phonemes.shape
    D = out_w.shape[1]
    V = phoneme_table.shape[0]
    N = B * T

    tc = pl.pallas_call(
        _tc_prelude_body,
        out_shape=[
            jax.ShapeDtypeStruct((V, D), jnp.float32),
            jax.ShapeDtypeStruct((B, D), jnp.float32),
            jax.ShapeDtypeStruct((8, D), jnp.float32),
            jax.ShapeDtypeStruct((N // 128, 128), jnp.int32),
            jax.ShapeDtypeStruct((N // 128, 128), jnp.float32),
        ],
    )
    tab, rr, fp, ph2, pn2 = tc(phoneme_table, speaker_table,
                               speakers.astype(jnp.int32).reshape(B, 1),
                               pitch_w, pitch_b.reshape(1, D), out_w,
                               out_b.reshape(1, D),
                               phonemes.astype(jnp.int32),
                               pitches.astype(jnp.float32))

    tpw = N // _NW
    mesh = plsc.VectorSubcoreMesh(core_axis_name="c", subcore_axis_name="s",
                                  num_cores=_NC, num_subcores=_NS)
    sc = pl.kernel(
        functools.partial(_sc_embed_body, tpw=tpw),
        out_type=jax.ShapeDtypeStruct((N, D), jnp.float32),
        mesh=mesh,
        scratch_types=[
            pltpu.VMEM((tpw // 128, 128), jnp.int32),
            pltpu.VMEM((tpw // 128, 128), jnp.float32),
            pltpu.VMEM((_RBUF, _CH, D), jnp.float32),
            pltpu.VMEM((_WBUF, _CH, D), jnp.float32),
            pltpu.VMEM((D,), jnp.float32),
            pltpu.VMEM((D,), jnp.float32),
            pltpu.SemaphoreType.DMA((_RBUF,)),
            pltpu.SemaphoreType.DMA((_WBUF,)),
            pltpu.SemaphoreType.DMA,
        ],
    )
    out = sc(ph2, pn2, tab, rr, fp)
    return out.reshape(B, T, D)


# speedup vs baseline: 2.0775x; 2.0775x over previous
import functools

import jax
import jax.numpy as jnp
from jax import lax
from jax.experimental import pallas as pl
from jax.experimental.pallas import tpu as pltpu
from jax.experimental.pallas import tpu_sc as plsc

_PITCH_MEAN = 150.0
_PITCH_STD = 50.0

_NC = 2
_NS = 16
_NW = _NC * _NS
_LANES = 16
_CH = 64
_RBUF = 3
_WBUF = 3


def _tc_prelude_body(pt_ref, st_ref, spk_ref, pw_ref, pb_ref, ow_ref, ob_ref,
                     ph_ref, pit_ref, tab_ref, rr_ref, fp_ref, ph2_ref,
                     pn2_ref):
    ow = ow_ref[...]
    prec = lax.Precision.HIGHEST
    tab_ref[...] = jnp.dot(pt_ref[...], ow, precision=prec,
                           preferred_element_type=jnp.float32)
    n_spk = st_ref.shape[0]
    b = spk_ref.shape[0]
    oh = (spk_ref[...] == lax.broadcasted_iota(jnp.int32, (b, n_spk), 1)
          ).astype(jnp.float32)
    srows = jnp.dot(oh, st_ref[...], precision=prec,
                    preferred_element_type=jnp.float32)
    const = jnp.dot(pb_ref[...], ow, precision=prec,
                    preferred_element_type=jnp.float32) + ob_ref[...]
    rr_ref[...] = jnp.dot(srows, ow, precision=prec,
                          preferred_element_type=jnp.float32) + const
    fp_ref[...] = jnp.broadcast_to(
        jnp.dot(pw_ref[...], ow, precision=prec,
                preferred_element_type=jnp.float32), fp_ref.shape)
    ph2_ref[...] = ph_ref[...].reshape(ph2_ref.shape)
    pn2_ref[...] = ((jnp.exp(pit_ref[...]) - _PITCH_MEAN)
                    / _PITCH_STD).reshape(pn2_ref.shape)


def _sc_embed_body(ph2_hbm, pn2_hbm, tab_hbm, rr_hbm, fp_hbm, out_hbm,
                   idx_v, pn_v, rows_v, obuf_v, fp_v, r_v, gsem, osem, csem,
                   tpw):
    cid = lax.axis_index("c")
    sid = lax.axis_index("s")
    wid = sid * _NC + cid
    base = wid * tpw
    n_chunks = tpw // _CH
    nrows = tpw // 128
    row0 = wid * nrows
    cpr = 128 // _CH
    b = base // (out_hbm.shape[0] // rr_hbm.shape[0])

    pltpu.sync_copy(ph2_hbm.at[pl.ds(row0, nrows)], idx_v)

    def gdesc(c):
        s = lax.rem(c, _RBUF)
        return pltpu.make_async_copy(
            tab_hbm.at[idx_v.at[c // cpr, pl.ds((c % cpr) * _CH, _CH)]],
            rows_v.at[s], gsem.at[s])

    def odesc(c):
        o = lax.rem(c, _WBUF)
        return pltpu.make_async_copy(
            obuf_v.at[o], out_hbm.at[pl.ds(base + c * _CH, _CH)], osem.at[o])

    gdesc(0).start()
    gdesc(1).start()
    gdesc(2).start()

    c_r = pltpu.async_copy(rr_hbm.at[b], r_v, csem)
    c_f = pltpu.async_copy(fp_hbm.at[0], fp_v, csem)
    c_p = pltpu.async_copy(pn2_hbm.at[pl.ds(row0, nrows)], pn_v, csem)
    c_r.wait()
    c_f.wait()
    c_p.wait()

    nvec = 256 // _LANES
    fp_regs = [fp_v[pl.ds(j * _LANES, _LANES)] for j in range(nvec)]
    r_regs = [r_v[pl.ds(j * _LANES, _LANES)] for j in range(nvec)]

    def chunk_body(c, carry):
        s = lax.rem(c, _RBUF)
        o = lax.rem(c, _WBUF)

        @pl.when(c >= _WBUF)
        def _drain():
            odesc(c - _WBUF).wait()

        gdesc(c).wait()

        def group(g, carry2):
            blk = pn_v[c // cpr, pl.ds((c % cpr) * _CH + g * _LANES, _LANES)]
            for k in range(_LANES):
                idx = jnp.full((_LANES, 1), k, jnp.int32)
                spl = lax.gather(
                    blk, idx,
                    lax.GatherDimensionNumbers(
                        offset_dims=(), collapsed_slice_dims=(0,),
                        start_index_map=(0,)),
                    slice_sizes=(1,),
                    mode=lax.GatherScatterMode.PROMISE_IN_BOUNDS)
                row = g * _LANES + k
                for j in range(nvec):
                    sl = pl.ds(j * _LANES, _LANES)
                    obuf_v[o, row, sl] = (rows_v[s, row, sl]
                                          + spl * fp_regs[j] + r_regs[j])
            return carry2

        lax.fori_loop(0, _CH // _LANES, group, 0)
        odesc(c).start()

        @pl.when(c + _RBUF < n_chunks)
        def _prefetch():
            gdesc(c + _RBUF).start()

        return carry

    lax.fori_loop(0, n_chunks, chunk_body, 0)
    for t in range(min(_WBUF, n_chunks)):
        odesc(n_chunks - 1 - t).wait()


def kernel(phonemes, pitches, speakers, phoneme_table, speaker_table,
           pitch_w, pitch_b, out_w, out_b):
    B, T = phonemes.shape
    D = out_w.shape[1]
    V = phoneme_table.shape[0]
    N = B * T

    tc = pl.pallas_call(
        _tc_prelude_body,
        out_shape=[
            jax.ShapeDtypeStruct((V, D), jnp.float32),
            jax.ShapeDtypeStruct((B, D), jnp.float32),
            jax.ShapeDtypeStruct((8, D), jnp.float32),
            jax.ShapeDtypeStruct((N // 128, 128), jnp.int32),
            jax.ShapeDtypeStruct((N // 128, 128), jnp.float32),
        ],
    )
    tab, rr, fp, ph2, pn2 = tc(phoneme_table, speaker_table,
                               speakers.astype(jnp.int32).reshape(B, 1),
                               pitch_w, pitch_b.reshape(1, D), out_w,
                               out_b.reshape(1, D),
                               phonemes.astype(jnp.int32),
                               pitches.astype(jnp.float32))

    tpw = N // _NW
    mesh = plsc.VectorSubcoreMesh(core_axis_name="c", subcore_axis_name="s",
                                  num_cores=_NC, num_subcores=_NS)
    sc = pl.kernel(
        functools.partial(_sc_embed_body, tpw=tpw),
        out_type=jax.ShapeDtypeStruct((N, D), jnp.float32),
        mesh=mesh,
        scratch_types=[
            pltpu.VMEM((tpw // 128, 128), jnp.int32),
            pltpu.VMEM((tpw // 128, 128), jnp.float32),
            pltpu.VMEM((_RBUF, _CH, D), jnp.float32),
            pltpu.VMEM((_WBUF, _CH, D), jnp.float32),
            pltpu.VMEM((D,), jnp.float32),
            pltpu.VMEM((D,), jnp.float32),
            pltpu.SemaphoreType.DMA((_RBUF,)),
            pltpu.SemaphoreType.DMA((_WBUF,)),
            pltpu.SemaphoreType.DMA,
        ],
    )
    out = sc(ph2, pn2, tab, rr, fp)
    return out.reshape(B, T, D)
